# baseline (speedup 1.0000x reference)
import functools

import jax
import jax.numpy as jnp
from jax import lax
from jax.experimental import pallas as pl
from jax.experimental.pallas import tpu as pltpu
from jax.experimental.pallas import tpu_sc as plsc

N = 10000
E = 320000
D = 128

NPAD = 10240
NT = NPAD // 16
ET_NORM = E // 16
K = 128
IGRP = 8
SLOW_CORE = 1
CS = 40
CF = 120
EPAD = 16 * (CS + CF) * K
ZROWS = 80
ROWBLK = 1000

_mesh = plsc.VectorSubcoreMesh(core_axis_name="c", subcore_axis_name="s")


def _rsqrt16(d):
    i = plsc.bitcast(d, jnp.int32)
    y = plsc.bitcast(jnp.int32(0x5F3759DF) - (i >> 1), jnp.float32)
    y = y * (1.5 - 0.5 * d * y * y)
    y = y * (1.5 - 0.5 * d * y * y)
    y = y * (1.5 - 0.5 * d * y * y)
    return jnp.where(d > 0.0, y, 0.0)


@functools.partial(
    pl.kernel,
    out_type=jax.ShapeDtypeStruct((4 * NPAD,), jnp.float32),
    mesh=_mesh,
    compiler_params=pltpu.CompilerParams(needs_layout_passes=False),
    scratch_types=[
        pltpu.VMEM((ET_NORM,), jnp.int32),
        pltpu.VMEM((ET_NORM,), jnp.int32),
        pltpu.VMEM((NPAD,), jnp.float32),
        pltpu.VMEM((NPAD,), jnp.float32),
        pltpu.VMEM((NT,), jnp.float32),
        pltpu.VMEM((NT,), jnp.float32),
        pltpu.VMEM_SHARED((16, NPAD), jnp.float32),
        pltpu.VMEM_SHARED((NPAD,), jnp.float32),
    ],
)
def _norm_kernel(rows_hbm, cols_hbm, out_hbm, rows_v, cols_v, hist_v,
                 dinv_v, acc_v, tmp_v, hists_sp, dinv_sp):
    c = lax.axis_index("c")
    s = lax.axis_index("s")
    ebase = s * ET_NORM
    nbase = s * NT
    _zeros16 = jnp.zeros((16,), jnp.float32)
    _ones16 = jnp.ones((16,), jnp.float32)

    pltpu.sync_copy(rows_hbm.at[pl.ds(c * E + ebase, ET_NORM)], rows_v)
    pltpu.sync_copy(cols_hbm.at[pl.ds(c * E + ebase, ET_NORM)], cols_v)

    def _zero_hist(i, _):
        hist_v[pl.ds(i * 16, 16)] = _zeros16
        return 0

    def _reduce_and_rsqrt(out_slot):
        pltpu.sync_copy(hists_sp.at[0, pl.ds(nbase, NT)], acc_v)

        def red_t(t, _):
            pltpu.sync_copy(hists_sp.at[t, pl.ds(nbase, NT)], tmp_v)

            def add_j(j, _):
                acc_v[pl.ds(j * 16, 16)] = (
                    acc_v[pl.ds(j * 16, 16)] + tmp_v[pl.ds(j * 16, 16)])
                return 0

            lax.fori_loop(0, NT // 16, add_j, 0)
            return 0

        lax.fori_loop(1, 16, red_t, 0)

        def rs_j(j, _):
            acc_v[pl.ds(j * 16, 16)] = _rsqrt16(acc_v[pl.ds(j * 16, 16)])
            return 0

        lax.fori_loop(0, NT // 16, rs_j, 0)
        pltpu.sync_copy(
            acc_v,
            out_hbm.at[pl.ds(c * (2 * NPAD) + out_slot * NPAD + nbase, NT)])

    lax.fori_loop(0, NPAD // 16, _zero_hist, 0)

    def hist_rows(i, _):
        idx = rows_v[pl.ds(i * 16, 16)]
        plsc.addupdate_scatter(hist_v, [idx], _ones16)
        return 0

    lax.fori_loop(0, ET_NORM // 16, hist_rows, 0)
    pltpu.sync_copy(hist_v, hists_sp.at[s])
    plsc.subcore_barrier()
    _reduce_and_rsqrt(0)
    pltpu.sync_copy(acc_v, dinv_sp.at[pl.ds(nbase, NT)])
    plsc.subcore_barrier()

    pltpu.sync_copy(dinv_sp, dinv_v)
    lax.fori_loop(0, NPAD // 16, _zero_hist, 0)

    def hist_cols(i, _):
        rv = rows_v[pl.ds(i * 16, 16)]
        cv = cols_v[pl.ds(i * 16, 16)]
        w = plsc.load_gather(dinv_v, [rv])
        plsc.addupdate_scatter(hist_v, [cv], w)
        return 0

    lax.fori_loop(0, ET_NORM // 16, hist_cols, 0)
    pltpu.sync_copy(hist_v, hists_sp.at[s])
    plsc.subcore_barrier()
    _reduce_and_rsqrt(1)


@functools.partial(
    pl.kernel,
    out_type=jax.ShapeDtypeStruct((2, NPAD, D), jnp.float32),
    mesh=_mesh,
    compiler_params=pltpu.CompilerParams(needs_layout_passes=False),
    scratch_types=[
        pltpu.VMEM((2, IGRP, K), jnp.int32),
        pltpu.VMEM((2, IGRP, K), jnp.int32),
        pltpu.VMEM((K, D), jnp.float32),
        pltpu.VMEM((K, D), jnp.float32),
        pltpu.VMEM_SHARED((NPAD, D), jnp.float32),
        pltpu.SemaphoreType.DMA,
        pltpu.SemaphoreType.DMA,
        pltpu.SemaphoreType.DMA,
    ],
)
def _spmm_kernel(g_hbm, colb_hbm, rowb_hbm, out_hbm, cidx_v, ridx_v,
                 rows0_v, rows1_v, acc_sp, gsem0, gsem1, isem):
    c = lax.axis_index("c")
    s = lax.axis_index("s")
    nbase = s * NT
    _zeros16 = jnp.zeros((16,), jnp.float32)
    rows = (rows0_v, rows1_v)
    gsems = (gsem0, gsem1)

    is_slow = c == SLOW_CORE
    chunk0 = s * (CS + CF) + jnp.where(is_slow, 0, CS)
    ngrp = jnp.where(is_slow, CS // IGRP, CF // IGRP)

    def idx_load_start(gg, slot):
        base = pl.multiple_of(chunk0 + gg * IGRP, 8)
        pltpu.async_copy(colb_hbm.at[pl.ds(base, IGRP)], cidx_v.at[slot], isem)
        pltpu.async_copy(rowb_hbm.at[pl.ds(base, IGRP)], ridx_v.at[slot], isem)

    def idx_load_wait(gg, slot):
        base = pl.multiple_of(chunk0 + gg * IGRP, 8)
        pltpu.make_async_copy(colb_hbm.at[pl.ds(base, IGRP)],
                              cidx_v.at[slot], isem).wait()
        pltpu.make_async_copy(rowb_hbm.at[pl.ds(base, IGRP)],
                              ridx_v.at[slot], isem).wait()

    idx_load_start(0, 0)

    def zb(i, _):
        def zb2(j, _):
            rows0_v[i, pl.ds(j * 16, 16)] = _zeros16
            return 0
        lax.fori_loop(0, D // 16, zb2, 0)
        return 0

    lax.fori_loop(0, ZROWS, zb, 0)

    def zcopy(j, _):
        pltpu.sync_copy(rows0_v.at[pl.ds(0, ZROWS)],
                        acc_sp.at[pl.ds(nbase + j * ZROWS, ZROWS)])
        return 0

    lax.fori_loop(0, NT // ZROWS, zcopy, 0)
    plsc.subcore_barrier()

    def group(g, _):
        slot = lax.rem(g, 2)
        idx_load_wait(g, slot)

        @pl.when(g + 1 < ngrp)
        def _():
            idx_load_start(g + 1, 1 - slot)

        pltpu.async_copy(g_hbm.at[cidx_v.at[slot, 0]], rows0_v, gsem0)
        for j in range(IGRP):
            b = j % 2
            pltpu.make_async_copy(g_hbm.at[cidx_v.at[slot, j]],
                                  rows[b], gsems[b]).wait()
            if j + 1 < IGRP:
                pltpu.async_copy(g_hbm.at[cidx_v.at[slot, j + 1]],
                                 rows[1 - b], gsems[1 - b])
            pltpu.sync_copy(rows[b], acc_sp.at[ridx_v.at[slot, j]],
                            add=True)
        return 0

    lax.fori_loop(0, ngrp, group, 0)
    plsc.subcore_barrier()

    pltpu.sync_copy(acc_sp.at[pl.ds(nbase, NT)],
                    out_hbm.at[c, pl.ds(nbase, NT)])


def _mm_scale_body(x_ref, w_ref, c_ref, o_ref):
    o_ref[...] = jnp.dot(x_ref[...], w_ref[...],
                         preferred_element_type=jnp.float32) * c_ref[...]


def _tc_mm_scale(x, W, cscale):
    return pl.pallas_call(
        _mm_scale_body,
        grid=(N // ROWBLK,),
        in_specs=[
            pl.BlockSpec((ROWBLK, D), lambda i: (i, 0)),
            pl.BlockSpec((D, D), lambda i: (0, 0)),
            pl.BlockSpec((ROWBLK, 1), lambda i: (i, 0)),
        ],
        out_specs=pl.BlockSpec((ROWBLK, D), lambda i: (i, 0)),
        out_shape=jax.ShapeDtypeStruct((N, D), jnp.float32),
    )(x, W, cscale)


def _mid_body(pa_ref, pb_ref, d_ref, b_ref, c_ref, w_ref, o_ref):
    t = (pa_ref[0] + pb_ref[0]) * d_ref[...] + b_ref[...]
    t = jnp.maximum(t, 0.0) * c_ref[...]
    o_ref[...] = jnp.dot(t, w_ref[...], preferred_element_type=jnp.float32)


def _tc_mid(p, dinv, b, cinv, W):
    return pl.pallas_call(
        _mid_body,
        grid=(N // ROWBLK,),
        in_specs=[
            pl.BlockSpec((1, ROWBLK, D), lambda i: (0, i, 0)),
            pl.BlockSpec((1, ROWBLK, D), lambda i: (1, i, 0)),
            pl.BlockSpec((ROWBLK, 1), lambda i: (i, 0)),
            pl.BlockSpec((1, D), lambda i: (0, 0)),
            pl.BlockSpec((ROWBLK, 1), lambda i: (i, 0)),
            pl.BlockSpec((D, D), lambda i: (0, 0)),
        ],
        out_specs=pl.BlockSpec((ROWBLK, D), lambda i: (i, 0)),
        out_shape=jax.ShapeDtypeStruct((N, D), jnp.float32),
    )(p, p, dinv, b, cinv, W)


def _fin_body(pa_ref, pb_ref, d_ref, b_ref, o_ref):
    t = (pa_ref[0] + pb_ref[0]) * d_ref[...] + b_ref[...]
    o_ref[...] = jnp.maximum(t, 0.0)


def _tc_final(p, dinv, b):
    return pl.pallas_call(
        _fin_body,
        grid=(N // ROWBLK,),
        in_specs=[
            pl.BlockSpec((1, ROWBLK, D), lambda i: (0, i, 0)),
            pl.BlockSpec((1, ROWBLK, D), lambda i: (1, i, 0)),
            pl.BlockSpec((ROWBLK, 1), lambda i: (i, 0)),
            pl.BlockSpec((1, D), lambda i: (0, 0)),
        ],
        out_specs=pl.BlockSpec((ROWBLK, D), lambda i: (i, 0)),
        out_shape=jax.ShapeDtypeStruct((N, D), jnp.float32),
    )(p, p, dinv, b)


def kernel(x, edge_index0, edge_index1, W1, b1, W2, b2):
    ei0 = edge_index0.astype(jnp.int32)
    ei1 = edge_index1.astype(jnp.int32)
    row0, col0 = ei0[0], ei0[1]
    row1, col1 = ei1[0], ei1[1]

    norm = _norm_kernel(jnp.concatenate([row0, row1]),
                        jnp.concatenate([col0, col1]))
    dinv0 = norm[0 * NPAD:0 * NPAD + N, None]
    cinv0 = norm[1 * NPAD:1 * NPAD + N, None]
    dinv1 = norm[2 * NPAD:2 * NPAD + N, None]
    cinv1 = norm[3 * NPAD:3 * NPAD + N, None]

    npad_e = EPAD - E
    pad_col = jnp.zeros((npad_e,), jnp.int32)
    pad_row = N + (jnp.arange(npad_e, dtype=jnp.int32) % (NPAD - N))
    colb0 = jnp.concatenate([col0, pad_col]).reshape(-1, K)
    rowb0 = jnp.concatenate([row0, pad_row]).reshape(-1, K)
    colb1 = jnp.concatenate([col1, pad_col]).reshape(-1, K)
    rowb1 = jnp.concatenate([row1, pad_row]).reshape(-1, K)

    g1 = _tc_mm_scale(x, W1, cinv0)
    p1 = _spmm_kernel(g1, colb0, rowb0)
    g2 = _tc_mid(p1, dinv0, b1.reshape(1, D), cinv1, W2)
    p2 = _spmm_kernel(g2, colb1, rowb1)
    return _tc_final(p2, dinv1, b2.reshape(1, D))


# speedup vs baseline: 1.1470x; 1.1470x over previous
import functools

import jax
import jax.numpy as jnp
from jax import lax
from jax.experimental import pallas as pl
from jax.experimental.pallas import tpu as pltpu
from jax.experimental.pallas import tpu_sc as plsc

N = 10000
E = 320000
D = 128

NPAD = 10240
NT = NPAD // 16
ET_NORM = E // 16
K = 128
IGRP = 8
CHUNKS = 80
NGRP = CHUNKS // IGRP
EPAD = 32 * CHUNKS * K
ZROWS = 80
ROWBLK = 1000

_mesh = plsc.VectorSubcoreMesh(core_axis_name="c", subcore_axis_name="s")


def _rsqrt16(d):
    i = plsc.bitcast(d, jnp.int32)
    y = plsc.bitcast(jnp.int32(0x5F3759DF) - (i >> 1), jnp.float32)
    y = y * (1.5 - 0.5 * d * y * y)
    y = y * (1.5 - 0.5 * d * y * y)
    y = y * (1.5 - 0.5 * d * y * y)
    return jnp.where(d > 0.0, y, 0.0)


@functools.partial(
    pl.kernel,
    out_type=jax.ShapeDtypeStruct((4 * NPAD,), jnp.float32),
    mesh=_mesh,
    compiler_params=pltpu.CompilerParams(needs_layout_passes=False),
    scratch_types=[
        pltpu.VMEM((ET_NORM,), jnp.int32),
        pltpu.VMEM((ET_NORM,), jnp.int32),
        pltpu.VMEM((NPAD,), jnp.float32),
        pltpu.VMEM((NPAD,), jnp.float32),
        pltpu.VMEM((NT,), jnp.float32),
        pltpu.VMEM((NT,), jnp.float32),
        pltpu.VMEM_SHARED((16, NPAD), jnp.float32),
        pltpu.VMEM_SHARED((NPAD,), jnp.float32),
    ],
)
def _norm_kernel(rows_hbm, cols_hbm, out_hbm, rows_v, cols_v, hist_v,
                 dinv_v, acc_v, tmp_v, hists_sp, dinv_sp):
    c = lax.axis_index("c")
    s = lax.axis_index("s")
    ebase = s * ET_NORM
    nbase = s * NT
    _zeros16 = jnp.zeros((16,), jnp.float32)
    _ones16 = jnp.ones((16,), jnp.float32)

    pltpu.sync_copy(rows_hbm.at[pl.ds(c * E + ebase, ET_NORM)], rows_v)
    pltpu.sync_copy(cols_hbm.at[pl.ds(c * E + ebase, ET_NORM)], cols_v)

    def _zero_hist(i, _):
        hist_v[pl.ds(i * 16, 16)] = _zeros16
        return 0

    def _reduce_and_rsqrt(out_slot):
        pltpu.sync_copy(hists_sp.at[0, pl.ds(nbase, NT)], acc_v)

        def red_t(t, _):
            pltpu.sync_copy(hists_sp.at[t, pl.ds(nbase, NT)], tmp_v)

            def add_j(j, _):
                acc_v[pl.ds(j * 16, 16)] = (
                    acc_v[pl.ds(j * 16, 16)] + tmp_v[pl.ds(j * 16, 16)])
                return 0

            lax.fori_loop(0, NT // 16, add_j, 0)
            return 0

        lax.fori_loop(1, 16, red_t, 0)

        def rs_j(j, _):
            acc_v[pl.ds(j * 16, 16)] = _rsqrt16(acc_v[pl.ds(j * 16, 16)])
            return 0

        lax.fori_loop(0, NT // 16, rs_j, 0)
        pltpu.sync_copy(
            acc_v,
            out_hbm.at[pl.ds(c * (2 * NPAD) + out_slot * NPAD + nbase, NT)])

    lax.fori_loop(0, NPAD // 16, _zero_hist, 0)

    def hist_rows(i, _):
        idx = rows_v[pl.ds(i * 16, 16)]
        plsc.addupdate_scatter(hist_v, [idx], _ones16)
        return 0

    lax.fori_loop(0, ET_NORM // 16, hist_rows, 0)
    pltpu.sync_copy(hist_v, hists_sp.at[s])
    plsc.subcore_barrier()
    _reduce_and_rsqrt(0)
    pltpu.sync_copy(acc_v, dinv_sp.at[pl.ds(nbase, NT)])
    plsc.subcore_barrier()

    pltpu.sync_copy(dinv_sp, dinv_v)
    lax.fori_loop(0, NPAD // 16, _zero_hist, 0)

    def hist_cols(i, _):
        rv = rows_v[pl.ds(i * 16, 16)]
        cv = cols_v[pl.ds(i * 16, 16)]
        w = plsc.load_gather(dinv_v, [rv])
        plsc.addupdate_scatter(hist_v, [cv], w)
        return 0

    lax.fori_loop(0, ET_NORM // 16, hist_cols, 0)
    pltpu.sync_copy(hist_v, hists_sp.at[s])
    plsc.subcore_barrier()
    _reduce_and_rsqrt(1)


@functools.partial(
    pl.kernel,
    out_type=jax.ShapeDtypeStruct((2, NPAD, D), jnp.float32),
    mesh=_mesh,
    compiler_params=pltpu.CompilerParams(needs_layout_passes=False),
    scratch_types=[
        pltpu.VMEM((2, IGRP, K), jnp.int32),
        pltpu.VMEM((2, IGRP, K), jnp.int32),
        pltpu.VMEM((K, D), jnp.float32),
        pltpu.VMEM((K, D), jnp.float32),
        pltpu.VMEM_SHARED((NPAD, D), jnp.float32),
        pltpu.SemaphoreType.DMA,
        pltpu.SemaphoreType.DMA,
        pltpu.SemaphoreType.DMA,
    ],
)
def _spmm_kernel(g_hbm, colb_hbm, rowb_hbm, out_hbm, cidx_v, ridx_v,
                 rows0_v, rows1_v, acc_sp, gsem0, gsem1, isem):
    c = lax.axis_index("c")
    s = lax.axis_index("s")
    w = s * 2 + c
    nbase = s * NT
    chunk0 = w * CHUNKS
    _zeros16 = jnp.zeros((16,), jnp.float32)
    rows = (rows0_v, rows1_v)
    gsems = (gsem0, gsem1)

    def idx_load_start(gg, slot):
        base = chunk0 + gg * IGRP
        pltpu.async_copy(colb_hbm.at[pl.ds(base, IGRP)], cidx_v.at[slot], isem)
        pltpu.async_copy(rowb_hbm.at[pl.ds(base, IGRP)], ridx_v.at[slot], isem)

    def idx_load_wait(gg, slot):
        base = chunk0 + gg * IGRP
        pltpu.make_async_copy(colb_hbm.at[pl.ds(base, IGRP)],
                              cidx_v.at[slot], isem).wait()
        pltpu.make_async_copy(rowb_hbm.at[pl.ds(base, IGRP)],
                              ridx_v.at[slot], isem).wait()

    idx_load_start(0, 0)

    def zb(i, _):
        def zb2(j, _):
            rows0_v[i, pl.ds(j * 16, 16)] = _zeros16
            return 0
        lax.fori_loop(0, D // 16, zb2, 0)
        return 0

    lax.fori_loop(0, ZROWS, zb, 0)

    def zcopy(j, _):
        pltpu.sync_copy(rows0_v.at[pl.ds(0, ZROWS)],
                        acc_sp.at[pl.ds(nbase + j * ZROWS, ZROWS)])
        return 0

    lax.fori_loop(0, NT // ZROWS, zcopy, 0)
    plsc.subcore_barrier()

    def group(g, _):
        slot = lax.rem(g, 2)
        idx_load_wait(g, slot)

        @pl.when(g + 1 < NGRP)
        def _():
            idx_load_start(g + 1, 1 - slot)

        pltpu.async_copy(g_hbm.at[cidx_v.at[slot, 0]], rows0_v, gsem0)
        for j in range(IGRP):
            b = j % 2
            pltpu.make_async_copy(g_hbm.at[cidx_v.at[slot, j]],
                                  rows[b], gsems[b]).wait()
            if j + 1 < IGRP:
                pltpu.async_copy(g_hbm.at[cidx_v.at[slot, j + 1]],
                                 rows[1 - b], gsems[1 - b])
            pltpu.sync_copy(rows[b], acc_sp.at[ridx_v.at[slot, j]],
                            add=True)
        return 0

    lax.fori_loop(0, NGRP, group, 0)
    plsc.subcore_barrier()

    pltpu.sync_copy(acc_sp.at[pl.ds(nbase, NT)],
                    out_hbm.at[c, pl.ds(nbase, NT)])


def _mm_scale_body(x_ref, w_ref, c_ref, o_ref):
    o_ref[...] = jnp.dot(x_ref[...], w_ref[...],
                         preferred_element_type=jnp.float32) * c_ref[...]


def _tc_mm_scale(x, W, cscale):
    return pl.pallas_call(
        _mm_scale_body,
        grid=(N // ROWBLK,),
        in_specs=[
            pl.BlockSpec((ROWBLK, D), lambda i: (i, 0)),
            pl.BlockSpec((D, D), lambda i: (0, 0)),
            pl.BlockSpec((ROWBLK, 1), lambda i: (i, 0)),
        ],
        out_specs=pl.BlockSpec((ROWBLK, D), lambda i: (i, 0)),
        out_shape=jax.ShapeDtypeStruct((N, D), jnp.float32),
    )(x, W, cscale)


def _mid_body(pa_ref, pb_ref, d_ref, b_ref, c_ref, w_ref, o_ref):
    t = (pa_ref[0] + pb_ref[0]) * d_ref[...] + b_ref[...]
    t = jnp.maximum(t, 0.0) * c_ref[...]
    o_ref[...] = jnp.dot(t, w_ref[...], preferred_element_type=jnp.float32)


def _tc_mid(p, dinv, b, cinv, W):
    return pl.pallas_call(
        _mid_body,
        grid=(N // ROWBLK,),
        in_specs=[
            pl.BlockSpec((1, ROWBLK, D), lambda i: (0, i, 0)),
            pl.BlockSpec((1, ROWBLK, D), lambda i: (1, i, 0)),
            pl.BlockSpec((ROWBLK, 1), lambda i: (i, 0)),
            pl.BlockSpec((1, D), lambda i: (0, 0)),
            pl.BlockSpec((ROWBLK, 1), lambda i: (i, 0)),
            pl.BlockSpec((D, D), lambda i: (0, 0)),
        ],
        out_specs=pl.BlockSpec((ROWBLK, D), lambda i: (i, 0)),
        out_shape=jax.ShapeDtypeStruct((N, D), jnp.float32),
    )(p, p, dinv, b, cinv, W)


def _fin_body(pa_ref, pb_ref, d_ref, b_ref, o_ref):
    t = (pa_ref[0] + pb_ref[0]) * d_ref[...] + b_ref[...]
    o_ref[...] = jnp.maximum(t, 0.0)


def _tc_final(p, dinv, b):
    return pl.pallas_call(
        _fin_body,
        grid=(N // ROWBLK,),
        in_specs=[
            pl.BlockSpec((1, ROWBLK, D), lambda i: (0, i, 0)),
            pl.BlockSpec((1, ROWBLK, D), lambda i: (1, i, 0)),
            pl.BlockSpec((ROWBLK, 1), lambda i: (i, 0)),
            pl.BlockSpec((1, D), lambda i: (0, 0)),
        ],
        out_specs=pl.BlockSpec((ROWBLK, D), lambda i: (i, 0)),
        out_shape=jax.ShapeDtypeStruct((N, D), jnp.float32),
    )(p, p, dinv, b)


def kernel(x, edge_index0, edge_index1, W1, b1, W2, b2):
    ei0 = edge_index0.astype(jnp.int32)
    ei1 = edge_index1.astype(jnp.int32)
    row0, col0 = ei0[0], ei0[1]
    row1, col1 = ei1[0], ei1[1]

    norm = _norm_kernel(jnp.concatenate([row0, row1]),
                        jnp.concatenate([col0, col1]))
    dinv0 = norm[0 * NPAD:0 * NPAD + N, None]
    cinv0 = norm[1 * NPAD:1 * NPAD + N, None]
    dinv1 = norm[2 * NPAD:2 * NPAD + N, None]
    cinv1 = norm[3 * NPAD:3 * NPAD + N, None]

    npad_e = EPAD - E
    pad_col = jnp.zeros((npad_e,), jnp.int32)
    pad_row = N + (jnp.arange(npad_e, dtype=jnp.int32) % (NPAD - N))
    colb0 = jnp.concatenate([col0, pad_col]).reshape(-1, K)
    rowb0 = jnp.concatenate([row0, pad_row]).reshape(-1, K)
    colb1 = jnp.concatenate([col1, pad_col]).reshape(-1, K)
    rowb1 = jnp.concatenate([row1, pad_row]).reshape(-1, K)

    g1 = _tc_mm_scale(x, W1, cinv0)
    p1 = _spmm_kernel(g1, colb0, rowb0)
    g2 = _tc_mid(p1, dinv0, b1.reshape(1, D), cinv1, W2)
    p2 = _spmm_kernel(g2, colb1, rowb1)
    return _tc_final(p2, dinv1, b2.reshape(1, D))
